# baseline (speedup 1.0000x reference)
import functools

import jax
import jax.numpy as jnp
from jax import lax
from jax.experimental import pallas as pl
from jax.experimental.pallas import tpu as pltpu
from jax.experimental.pallas import tpu_sc as plsc

B, N, C = 8, 65536, 64
M, K = 32768, 27

NC, NS, L = 2, 16, 16
NW = NC * NS
R = (M * K) // NW
CR = 512
G = CR // 128
NCHUNK = R // CR
M_PER_W = M // NW


def _sparse_select(features_flat, batches, offsets_flat):
    mesh = plsc.VectorSubcoreMesh(core_axis_name="c", subcore_axis_name="s")

    @functools.partial(
        pl.kernel,
        mesh=mesh,
        compiler_params=pltpu.CompilerParams(use_tc_tiling_on_sc=False),
        out_type=jax.ShapeDtypeStruct((M * K, C), jnp.float32),
        scratch_types=[
            pltpu.VMEM((M_PER_W + 16,), jnp.int32),
            pltpu.VMEM((CR,), jnp.int32),
            pltpu.VMEM((G, 128), jnp.int32),
            pltpu.VMEM((CR, C), jnp.float32),
            pltpu.SemaphoreType.DMA,
        ],
    )
    def body(feat_hbm, batches_hbm, offs_hbm, out_hbm,
             bat_v, offs_v, idx_v, rows_v, sem):
        wid = lax.axis_index("s") * NC + lax.axis_index("c")
        wbase = wid * R
        pltpu.sync_copy(batches_hbm.at[pl.ds(wid * M_PER_W, M_PER_W)],
                        bat_v.at[pl.ds(0, M_PER_W)])

        def chunk(c, carry):
            base = c * CR
            pltpu.sync_copy(offs_hbm.at[pl.ds(wbase + base, CR)], offs_v)
            iota16 = lax.iota(jnp.int32, 16)
            for g in range(G):
                for j in range(8):
                    jj = g * 8 + j
                    pos0 = base + jj * 16
                    m0 = pos0 // K
                    r0 = pos0 - m0 * K
                    bwin = bat_v[pl.ds(m0, 16)]
                    rel = jnp.where(iota16 >= K - r0, 1, 0).astype(jnp.int32)
                    b = lax.gather(
                        bwin, rel[:, None],
                        dimension_numbers=lax.GatherDimensionNumbers(
                            offset_dims=(), collapsed_slice_dims=(0,),
                            start_index_map=(0,)),
                        slice_sizes=(1,),
                        mode=lax.GatherScatterMode.PROMISE_IN_BOUNDS)
                    off = offs_v[pl.ds(jj * 16, 16)]
                    idx_v[g, pl.ds(j * 16, 16)] = b * N + off
            copies = [
                pltpu.async_copy(
                    feat_hbm.at[idx_v.at[g]],
                    rows_v.at[pl.ds(g * 128, 128)],
                    sem,
                )
                for g in range(G)
            ]
            for cp in copies:
                cp.wait()
            pltpu.sync_copy(rows_v, out_hbm.at[pl.ds(wbase + base, CR)])
            return carry

        lax.fori_loop(0, NCHUNK, chunk, 0)

    return body(features_flat, batches, offsets_flat)


def kernel(features, batches, offsets):
    features_flat = features.reshape(B * N, C)
    offsets_flat = offsets.astype(jnp.int32).reshape(M * K)
    out = _sparse_select(features_flat, batches.astype(jnp.int32), offsets_flat)
    return out.reshape(M, K, C)


# speedup vs baseline: 1.0633x; 1.0633x over previous
import functools

import jax
import jax.numpy as jnp
from jax import lax
from jax.experimental import pallas as pl
from jax.experimental.pallas import tpu as pltpu
from jax.experimental.pallas import tpu_sc as plsc

B, N, C = 8, 65536, 64
M, K = 32768, 27

NC, NS, L = 2, 16, 16
NW = NC * NS
R = (M * K) // NW
CR = 512
G = CR // 128
NCHUNK = R // CR
M_PER_W = M // NW


def _sparse_select(features_flat, batches, offsets_flat):
    mesh = plsc.VectorSubcoreMesh(core_axis_name="c", subcore_axis_name="s")

    @functools.partial(
        pl.kernel,
        mesh=mesh,
        compiler_params=pltpu.CompilerParams(use_tc_tiling_on_sc=False),
        out_type=jax.ShapeDtypeStruct((M * K, C), jnp.float32),
        scratch_types=[
            pltpu.VMEM((M_PER_W + 16,), jnp.int32),
            pltpu.VMEM((R,), jnp.int32),
            pltpu.VMEM((G, 128), jnp.int32),
            pltpu.VMEM((G, 128), jnp.int32),
            pltpu.VMEM((CR, C), jnp.float32),
            pltpu.VMEM((CR, C), jnp.float32),
            pltpu.SemaphoreType.DMA,
            pltpu.SemaphoreType.DMA,
            pltpu.SemaphoreType.DMA,
            pltpu.SemaphoreType.DMA,
        ],
    )
    def body(feat_hbm, batches_hbm, offs_hbm, out_hbm,
             bat_v, offs_v, idx0, idx1, rows0, rows1,
             sem_g0, sem_g1, sem_w0, sem_w1):
        wid = lax.axis_index("s") * NC + lax.axis_index("c")
        wbase = wid * R
        pltpu.sync_copy(batches_hbm.at[pl.ds(wid * M_PER_W, M_PER_W)],
                        bat_v.at[pl.ds(0, M_PER_W)])
        pltpu.sync_copy(offs_hbm.at[pl.ds(wbase, R)], offs_v)
        iota16 = lax.iota(jnp.int32, 16)

        def compute_idx(c, idx_v):
            base = c * CR
            for g in range(G):
                for j in range(8):
                    jj = g * 8 + j
                    pos0 = base + jj * 16
                    m0 = pos0 // K
                    r0 = pos0 - m0 * K
                    bwin = bat_v[pl.ds(m0, 16)]
                    rel = jnp.where(iota16 >= K - r0, 1, 0).astype(jnp.int32)
                    b = lax.gather(
                        bwin, rel[:, None],
                        dimension_numbers=lax.GatherDimensionNumbers(
                            offset_dims=(), collapsed_slice_dims=(0,),
                            start_index_map=(0,)),
                        slice_sizes=(1,),
                        mode=lax.GatherScatterMode.PROMISE_IN_BOUNDS)
                    off = offs_v[pl.ds(pos0, 16)]
                    idx_v[g, pl.ds(j * 16, 16)] = b * N + off

        def fire_gathers(idx_v, rows_v, sem):
            return [
                pltpu.async_copy(
                    feat_hbm.at[idx_v.at[g]],
                    rows_v.at[pl.ds(g * 128, 128)],
                    sem,
                )
                for g in range(G)
            ]

        def fire_wb(c, rows_v, sem):
            pltpu.async_copy(rows_v, out_hbm.at[pl.ds(wbase + c * CR, CR)], sem)

        def wb_wait(rows_v, sem):
            pltpu.make_async_copy(rows_v, out_hbm.at[pl.ds(0, CR)], sem).wait()

        def pair(i, carry):
            a = 2 * i
            b = a + 1
            compute_idx(a, idx0)
            compute_idx(b, idx1)

            @pl.when(i > 0)
            def _():
                wb_wait(rows0, sem_w0)
            ga = fire_gathers(idx0, rows0, sem_g0)

            @pl.when(i > 0)
            def _():
                wb_wait(rows1, sem_w1)
            gb = fire_gathers(idx1, rows1, sem_g1)

            for h in ga:
                h.wait()
            fire_wb(a, rows0, sem_w0)
            for h in gb:
                h.wait()
            fire_wb(b, rows1, sem_w1)
            return carry

        lax.fori_loop(0, NCHUNK // 2, pair, 0)
        wb_wait(rows0, sem_w0)
        wb_wait(rows1, sem_w1)

    return body(features_flat, batches, offsets_flat)


def kernel(features, batches, offsets):
    features_flat = features.reshape(B * N, C)
    offsets_flat = offsets.astype(jnp.int32).reshape(M * K)
    out = _sparse_select(features_flat, batches.astype(jnp.int32), offsets_flat)
    return out.reshape(M, K, C)


# speedup vs baseline: 1.1150x; 1.0486x over previous
import functools

import jax
import jax.numpy as jnp
from jax import lax
from jax.experimental import pallas as pl
from jax.experimental.pallas import tpu as pltpu
from jax.experimental.pallas import tpu_sc as plsc

B, N, C = 8, 65536, 64
M, K = 32768, 27

NC, NS, L = 2, 16, 16
NW = NC * NS
M_PER_W = M // NW
CR = 512
HALVES = M_PER_W // CR
G = CR // 128


def _sparse_select(features_flat, batches, offsets_t):
    mesh = plsc.VectorSubcoreMesh(core_axis_name="c", subcore_axis_name="s")

    @functools.partial(
        pl.kernel,
        mesh=mesh,
        compiler_params=pltpu.CompilerParams(use_tc_tiling_on_sc=False),
        out_type=jax.ShapeDtypeStruct((K * M, C), jnp.float32),
        scratch_types=[
            pltpu.VMEM((M_PER_W,), jnp.int32),
            pltpu.VMEM((K, M_PER_W), jnp.int32),
            pltpu.VMEM((G, 128), jnp.int32),
            pltpu.VMEM((G, 128), jnp.int32),
            pltpu.VMEM((CR, C), jnp.float32),
            pltpu.VMEM((CR, C), jnp.float32),
            pltpu.SemaphoreType.DMA,
            pltpu.SemaphoreType.DMA,
            pltpu.SemaphoreType.DMA,
            pltpu.SemaphoreType.DMA,
        ],
    )
    def body(feat_hbm, batches_hbm, offs_hbm, out_hbm,
             bat_v, offs_v, idx0, idx1, rows0, rows1,
             sem_g0, sem_g1, sem_w0, sem_w1):
        wid = lax.axis_index("s") * NC + lax.axis_index("c")
        mw0 = wid * M_PER_W
        pltpu.sync_copy(batches_hbm.at[pl.ds(mw0, M_PER_W)], bat_v)
        pltpu.sync_copy(offs_hbm.at[:, pl.ds(mw0, M_PER_W)], offs_v)

        def compute_idx(k, h, idx_v):
            for g in range(G):
                for j in range(8):
                    m_loc = h * CR + (g * 8 + j) * 16
                    b = bat_v[pl.ds(m_loc, 16)]
                    off = offs_v[k, pl.ds(m_loc, 16)]
                    idx_v[g, pl.ds(j * 16, 16)] = b * N + off

        def fire_gathers(idx_v, rows_v, sem):
            return [
                pltpu.async_copy(
                    feat_hbm.at[idx_v.at[g]],
                    rows_v.at[pl.ds(g * 128, 128)],
                    sem,
                )
                for g in range(G)
            ]

        def fire_wb(k, h, rows_v, sem):
            row0 = k * M + mw0 + h * CR
            pltpu.async_copy(rows_v, out_hbm.at[pl.ds(row0, CR)], sem)

        def wb_wait(rows_v, sem):
            pltpu.make_async_copy(rows_v, out_hbm.at[pl.ds(0, CR)], sem).wait()

        def per_k(k, carry):
            compute_idx(k, 0, idx0)
            compute_idx(k, 1, idx1)

            @pl.when(k > 0)
            def _():
                wb_wait(rows0, sem_w0)
            ga = fire_gathers(idx0, rows0, sem_g0)

            @pl.when(k > 0)
            def _():
                wb_wait(rows1, sem_w1)
            gb = fire_gathers(idx1, rows1, sem_g1)

            for h in ga:
                h.wait()
            fire_wb(k, 0, rows0, sem_w0)
            for h in gb:
                h.wait()
            fire_wb(k, 1, rows1, sem_w1)
            return carry

        lax.fori_loop(0, K, per_k, 0)
        wb_wait(rows0, sem_w0)
        wb_wait(rows1, sem_w1)

    return body(features_flat, batches, offsets_t)


def kernel(features, batches, offsets):
    features_flat = features.reshape(B * N, C)
    offsets_t = offsets.astype(jnp.int32).T
    out = _sparse_select(features_flat, batches.astype(jnp.int32), offsets_t)
    return out.reshape(K, M, C).transpose(1, 0, 2)
